# Initial kernel scaffold; baseline (speedup 1.0000x reference)
import jax
import jax.numpy as jnp
from jax.experimental import pallas as pl


def kernel(user_indices, item_indices, edge_index, user_table, item_table):
    raise NotImplementedError("write your pallas kernel here")


# speedup vs baseline: 17.8544x; 17.8544x over previous
import jax
import jax.numpy as jnp
from jax import lax
from jax.experimental import pallas as pl
from jax.experimental.pallas import tpu as pltpu
from jax.experimental.pallas import tpu_sc as plsc

NUSERS = 50000
NITEMS = 50000
NNODES = 100000
EMBH = 16
NEDGES = 1600000
NBATCH = 16384

NC = 2
NS = 16
NW = NC * NS
LANES = 16

ECHUNK = 1000
EPAD = 1008
EPT = NEDGES // NS
NECH = EPT // ECHUNK

NCHUNK = 1000
NFULL = NCHUNK // LANES
NTAIL = NCHUNK - NFULL * LANES
NNCH = NNODES // NCHUNK
NNCH_HALF = NNCH // NC

BPW = NBATCH // NW
CHB = 256

_mesh = plsc.VectorSubcoreMesh(core_axis_name="c", subcore_axis_name="s")

_f32 = jnp.float32
_i32 = jnp.int32


def _ceil_div(a, b):
    return -(-a // b)


def _lane_bcast(vec, j):
    return vec.at[jnp.full((LANES,), j, _i32)].get(mode="promise_in_bounds")


def _deg_body(dst_hbm, inv_hbm, deg_sh, idx_v, ones_v, buf_v):
    c = lax.axis_index("c")
    s = lax.axis_index("s")

    def fill1(i, carry):
        ones_v[pl.ds(i * LANES, LANES)] = jnp.full((LANES,), 1.0, _f32)
        return carry

    lax.fori_loop(0, EPAD // LANES, fill1, 0)

    def fill0(i, carry):
        buf_v[pl.ds(i * LANES, LANES)] = jnp.zeros((LANES,), _f32)
        return carry

    lax.fori_loop(0, EPAD // LANES, fill0, 0)

    for k in range(_ceil_div(NNCH, NS)):
        cid = s + k * NS

        @pl.when(cid < NNCH)
        def _():
            pltpu.sync_copy(buf_v.at[pl.ds(0, NCHUNK)],
                            deg_sh.at[pl.ds(cid * NCHUNK, NCHUNK)])

    plsc.subcore_barrier()

    def edge_body(j, carry):
        off = s * EPT + j * ECHUNK
        pltpu.sync_copy(dst_hbm.at[pl.ds(off, ECHUNK)], idx_v)
        pltpu.sync_copy(ones_v.at[pl.ds(0, ECHUNK)], deg_sh.at[idx_v],
                        add=True)
        return carry

    lax.fori_loop(0, NECH, edge_body, 0)
    plsc.subcore_barrier()

    def inv_chunk(cid):
        pltpu.sync_copy(deg_sh.at[pl.ds(cid * NCHUNK, NCHUNK)],
                        buf_v.at[pl.ds(0, NCHUNK)])

        def vloop(i, carry):
            v = buf_v[pl.ds(i * LANES, LANES)]
            buf_v[pl.ds(i * LANES, LANES)] = 1.0 / jnp.maximum(v, 1.0)
            return carry

        lax.fori_loop(0, EPAD // LANES, vloop, 0)
        pltpu.sync_copy(buf_v.at[pl.ds(0, NCHUNK)],
                        inv_hbm.at[pl.ds(cid * NCHUNK, NCHUNK)])

    for k in range(_ceil_div(NNCH_HALF, NS)):
        cid_local = s + k * NS

        @pl.when(cid_local < NNCH_HALF)
        def _():
            inv_chunk(c * NNCH_HALF + cid_local)


_deg_kernel = pl.kernel(
    _deg_body,
    out_type=jax.ShapeDtypeStruct((NNODES,), _f32),
    mesh=_mesh,
    compiler_params=pltpu.CompilerParams(use_tc_tiling_on_sc=False, needs_layout_passes=False),
    scratch_types=[
        pltpu.VMEM_SHARED((NNODES,), _f32),
        pltpu.VMEM((ECHUNK,), _i32),
        pltpu.VMEM((EPAD,), _f32),
        pltpu.VMEM((EPAD,), _f32),
    ],
)


def _layer_body(tabA, tabB, src_hbm, dst_hbm, inv_hbm,
                outA, outB, agg_sh, srcv, dstv, rows_v, inv_v, sem):
    c = lax.axis_index("c")
    s = lax.axis_index("s")

    def half(tab, out):
        def fill0(r, carry):
            rows_v[r, :] = jnp.zeros((LANES,), _f32)
            return carry

        lax.fori_loop(0, NCHUNK, fill0, 0)

        for k in range(_ceil_div(NNCH, NS)):
            cid = s + k * NS

            @pl.when(cid < NNCH)
            def _():
                pltpu.sync_copy(rows_v,
                                agg_sh.at[pl.ds(cid * NCHUNK, NCHUNK)])

        plsc.subcore_barrier()

        def edge_body(j, carry):
            off = s * EPT + j * ECHUNK
            pltpu.sync_copy(src_hbm.at[pl.ds(off, ECHUNK)], srcv)
            pltpu.sync_copy(dst_hbm.at[pl.ds(off, ECHUNK)], dstv)
            pltpu.async_copy(tab.at[srcv], rows_v, sem).wait()
            pltpu.sync_copy(rows_v, agg_sh.at[dstv], add=True)
            return carry

        lax.fori_loop(0, NECH, edge_body, 0)
        plsc.subcore_barrier()

        def scale_chunk(cid):
            pltpu.sync_copy(agg_sh.at[pl.ds(cid * NCHUNK, NCHUNK)], rows_v)
            pltpu.sync_copy(inv_hbm.at[pl.ds(cid * NCHUNK, NCHUNK)],
                            inv_v.at[pl.ds(0, NCHUNK)])

            def srow(i, carry):
                iv = inv_v[pl.ds(i * LANES, LANES)]
                for j in range(LANES):
                    m = _lane_bcast(iv, j)
                    r = i * LANES + j
                    rows_v[r, :] = rows_v[r, :] * m
                return carry

            lax.fori_loop(0, NFULL, srow, 0)
            iv_t = inv_v[pl.ds(NFULL * LANES, LANES)]
            for j in range(NTAIL):
                m = _lane_bcast(iv_t, j)
                r = NFULL * LANES + j
                rows_v[r, :] = rows_v[r, :] * m
            pltpu.sync_copy(rows_v, out.at[pl.ds(cid * NCHUNK, NCHUNK)])

        for k in range(_ceil_div(NNCH, NS)):
            cid = s + k * NS

            @pl.when(cid < NNCH)
            def _():
                scale_chunk(cid)

    @pl.when(c == 0)
    def _():
        half(tabA, outA)

    @pl.when(c == 1)
    def _():
        half(tabB, outB)


_layer_kernel = pl.kernel(
    _layer_body,
    out_type=(
        jax.ShapeDtypeStruct((NNODES, EMBH), _f32),
        jax.ShapeDtypeStruct((NNODES, EMBH), _f32),
    ),
    mesh=_mesh,
    compiler_params=pltpu.CompilerParams(use_tc_tiling_on_sc=False, needs_layout_passes=False),
    scratch_types=[
        pltpu.VMEM_SHARED((NNODES, EMBH), _f32),
        pltpu.VMEM((ECHUNK,), _i32),
        pltpu.VMEM((ECHUNK,), _i32),
        pltpu.VMEM((ECHUNK, EMBH), _f32),
        pltpu.VMEM((EPAD,), _f32),
        pltpu.SemaphoreType.DMA,
    ],
)


def _final_body(A0, B0, A1, B1, A2, B2, A3, B3, uidx_hbm, iidx_hbm,
                pred_hbm, reg_hbm,
                ids_v, g00, g01, g10, g11, g20, g21, g30, g31,
                pred_v, rbuf_v, sem):
    c = lax.axis_index("c")
    s = lax.axis_index("s")
    w = s * NC + c
    base = w * BPW

    iota = lax.broadcasted_iota(_i32, (LANES,), 0)
    bufs = ((g00, g01), (g10, g11), (g20, g21), (g30, g31))
    tabs = ((A0, B0), (A1, B1), (A2, B2), (A3, B3))

    regacc = jnp.zeros((LANES,), _f32)
    for cc in range(BPW // CHB):
        cbase = base + cc * CHB
        pltpu.sync_copy(uidx_hbm.at[pl.ds(cbase, CHB)],
                        ids_v.at[pl.ds(0, CHB)])
        pltpu.sync_copy(iidx_hbm.at[pl.ds(cbase, CHB)],
                        ids_v.at[pl.ds(CHB, CHB)])

        def addoff(i, carry):
            v = ids_v[pl.ds(CHB + i * LANES, LANES)]
            ids_v[pl.ds(CHB + i * LANES, LANES)] = v + NUSERS
            return carry

        lax.fori_loop(0, CHB // LANES, addoff, 0)

        for l in range(4):
            pltpu.async_copy(tabs[l][0].at[ids_v], bufs[l][0], sem).wait()
            pltpu.async_copy(tabs[l][1].at[ids_v], bufs[l][1], sem).wait()

        def group_body(g, racc):
            pvec = jnp.zeros((LANES,), _f32)
            for j in range(LANES):
                b = g * LANES + j
                uA = g00[b, :] + g10[b, :] + g20[b, :] + g30[b, :]
                uB = g01[b, :] + g11[b, :] + g21[b, :] + g31[b, :]
                iA = (g00[CHB + b, :] + g10[CHB + b, :]
                      + g20[CHB + b, :] + g30[CHB + b, :])
                iB = (g01[CHB + b, :] + g11[CHB + b, :]
                      + g21[CHB + b, :] + g31[CHB + b, :])
                t = uA * iA + uB * iB
                p = jnp.sum(t) * (1.0 / 16.0)
                onehot = (iota == j).astype(_f32)
                pvec = pvec + p * onehot
                u0A = g00[b, :]
                u0B = g01[b, :]
                i0A = g00[CHB + b, :]
                i0B = g01[CHB + b, :]
                racc = (racc + u0A * u0A + u0B * u0B
                        + i0A * i0A + i0B * i0B)
            pred_v[pl.ds(cc * CHB + g * LANES, LANES)] = pvec
            return racc

        regacc = lax.fori_loop(0, CHB // LANES, group_body, regacc)

    pltpu.sync_copy(pred_v, pred_hbm.at[pl.ds(base, BPW)])
    rbuf_v[0, :] = regacc
    pltpu.sync_copy(rbuf_v, reg_hbm.at[pl.ds(w, 1)])


_final_kernel = pl.kernel(
    _final_body,
    out_type=(
        jax.ShapeDtypeStruct((NBATCH,), _f32),
        jax.ShapeDtypeStruct((NW, LANES), _f32),
    ),
    mesh=_mesh,
    compiler_params=pltpu.CompilerParams(use_tc_tiling_on_sc=False, needs_layout_passes=False),
    scratch_types=[
        pltpu.VMEM((2 * CHB,), _i32),
        pltpu.VMEM((2 * CHB, EMBH), _f32),
        pltpu.VMEM((2 * CHB, EMBH), _f32),
        pltpu.VMEM((2 * CHB, EMBH), _f32),
        pltpu.VMEM((2 * CHB, EMBH), _f32),
        pltpu.VMEM((2 * CHB, EMBH), _f32),
        pltpu.VMEM((2 * CHB, EMBH), _f32),
        pltpu.VMEM((2 * CHB, EMBH), _f32),
        pltpu.VMEM((2 * CHB, EMBH), _f32),
        pltpu.VMEM((BPW,), _f32),
        pltpu.VMEM((1, LANES), _f32),
        pltpu.SemaphoreType.DMA,
    ],
)


def kernel(user_indices, item_indices, edge_index, user_table, item_table):
    A0 = jnp.concatenate([user_table[:, :EMBH], item_table[:, :EMBH]], axis=0)
    B0 = jnp.concatenate([user_table[:, EMBH:], item_table[:, EMBH:]], axis=0)
    src = edge_index[0].astype(_i32)
    dst = edge_index[1].astype(_i32)

    inv = _deg_kernel(dst)
    A1, B1 = _layer_kernel(A0, B0, src, dst, inv)
    A2, B2 = _layer_kernel(A1, B1, src, dst, inv)
    A3, B3 = _layer_kernel(A2, B2, src, dst, inv)
    pred, regpart = _final_kernel(A0, B0, A1, B1, A2, B2, A3, B3,
                                  user_indices.astype(_i32),
                                  item_indices.astype(_i32))
    reg_loss = 0.5 * jnp.sum(regpart) / float(NBATCH)
    return pred, reg_loss


# speedup vs baseline: 20.6599x; 1.1571x over previous
import jax
import jax.numpy as jnp
from jax import lax
from jax.experimental import pallas as pl
from jax.experimental.pallas import tpu as pltpu
from jax.experimental.pallas import tpu_sc as plsc

NUSERS = 50000
NITEMS = 50000
NNODES = 100000
EMBH = 16
NEDGES = 1600000
NBATCH = 16384

NC = 2
NS = 16
NW = NC * NS
LANES = 16

EPT = NEDGES // NS

ECHUNK = 400
NECH = EPT // ECHUNK
MAINT = (NECH - 2) // 4
NCHUNK = 400
NFULL = NCHUNK // LANES
NNCH = NNODES // NCHUNK
NROUND = -(-NNCH // NS)

DCHUNK = 2000
DNECH = EPT // DCHUNK
DNCH = NNODES // DCHUNK
DNCH_HALF = DNCH // NC
DROUND = -(-DNCH // NS)
DROUND_HALF = -(-DNCH_HALF // NS)

BPW = NBATCH // NW
CHB = 256

_mesh = plsc.VectorSubcoreMesh(core_axis_name="c", subcore_axis_name="s")
_params = pltpu.CompilerParams(use_tc_tiling_on_sc=False,
                               needs_layout_passes=False)

_f32 = jnp.float32
_i32 = jnp.int32


def _lane_bcast(vec, j):
    return vec.at[jnp.full((LANES,), j, _i32)].get(mode="promise_in_bounds")


def _deg_body(dst_hbm, inv_hbm, deg_sh, idx_v, ones_v, buf_v):
    c = lax.axis_index("c")
    s = lax.axis_index("s")

    def fill1(i, carry):
        ones_v[pl.ds(i * LANES, LANES)] = jnp.full((LANES,), 1.0, _f32)
        return carry

    lax.fori_loop(0, DCHUNK // LANES, fill1, 0)

    def fill0(i, carry):
        buf_v[pl.ds(i * LANES, LANES)] = jnp.zeros((LANES,), _f32)
        return carry

    lax.fori_loop(0, DCHUNK // LANES, fill0, 0)

    def zloop(k, carry):
        cid = s + k * NS

        @pl.when(cid < DNCH)
        def _():
            pltpu.sync_copy(buf_v, deg_sh.at[pl.ds(cid * DCHUNK, DCHUNK)])

        return carry

    lax.fori_loop(0, DROUND, zloop, 0)
    plsc.subcore_barrier()

    def edge_body(j, carry):
        off = s * EPT + j * DCHUNK
        pltpu.sync_copy(dst_hbm.at[pl.ds(off, DCHUNK)], idx_v)
        pltpu.sync_copy(ones_v, deg_sh.at[idx_v], add=True)
        return carry

    lax.fori_loop(0, DNECH, edge_body, 0)
    plsc.subcore_barrier()

    def iloop(k, carry):
        cid_local = s + k * NS

        @pl.when(cid_local < DNCH_HALF)
        def _():
            cid = c * DNCH_HALF + cid_local
            pltpu.sync_copy(deg_sh.at[pl.ds(cid * DCHUNK, DCHUNK)], buf_v)

            def vloop(i, carry2):
                v = buf_v[pl.ds(i * LANES, LANES)]
                buf_v[pl.ds(i * LANES, LANES)] = 1.0 / jnp.maximum(v, 1.0)
                return carry2

            lax.fori_loop(0, DCHUNK // LANES, vloop, 0)
            pltpu.sync_copy(buf_v, inv_hbm.at[pl.ds(cid * DCHUNK, DCHUNK)])

        return carry

    lax.fori_loop(0, DROUND_HALF, iloop, 0)


_deg_kernel = pl.kernel(
    _deg_body,
    out_type=jax.ShapeDtypeStruct((NNODES,), _f32),
    mesh=_mesh,
    compiler_params=_params,
    scratch_types=[
        pltpu.VMEM_SHARED((NNODES,), _f32),
        pltpu.VMEM((DCHUNK,), _i32),
        pltpu.VMEM((DCHUNK,), _f32),
        pltpu.VMEM((DCHUNK,), _f32),
    ],
)


def _layer_body(tabA, tabB, eidx_hbm, inv_hbm, outA, outB,
                agg_sh, eb0, eb1, eb2, eb3, rows0, rows1, inv_v,
                is0, is1, is2, is3, gsem, ssem):
    c = lax.axis_index("c")
    s = lax.axis_index("s")
    ebufs = (eb0, eb1, eb2, eb3)
    isems = (is0, is1, is2, is3)
    rows = (rows0, rows1)

    def half(tab, out):
        def fillz(r, carry):
            rows0[r, :] = jnp.zeros((LANES,), _f32)
            return carry

        lax.fori_loop(0, NCHUNK, fillz, 0)

        def zloop(k, carry):
            cid = s + k * NS

            @pl.when(cid < NNCH)
            def _():
                pltpu.sync_copy(rows0, agg_sh.at[pl.ds(cid * NCHUNK, NCHUNK)])

            return carry

        lax.fori_loop(0, NROUND, zloop, 0)
        plsc.subcore_barrier()

        def istart(j, slot):
            off = s * EPT + j * ECHUNK
            pltpu.async_copy(eidx_hbm.at[:, pl.ds(off, ECHUNK)],
                             ebufs[slot], isems[slot])

        def iwait(slot):
            pltpu.make_async_copy(eidx_hbm.at[:, pl.ds(0, ECHUNK)],
                                  ebufs[slot], isems[slot]).wait()

        def gstart(slot):
            pltpu.async_copy(tab.at[ebufs[slot].at[0]], rows[slot % 2], gsem)

        def gwait(slot):
            pltpu.make_async_copy(tab.at[ebufs[slot].at[0]],
                                  rows[slot % 2], gsem).wait()

        def sstart(slot):
            pltpu.async_copy(rows[slot % 2], agg_sh.at[ebufs[slot].at[1]],
                             ssem, add=True)

        def swait(slot):
            pltpu.make_async_copy(rows[slot % 2],
                                  agg_sh.at[ebufs[slot].at[1]],
                                  ssem).wait()

        istart(0, 0)
        istart(1, 1)
        istart(2, 2)
        iwait(0)
        gstart(0)

        def edge_round(t, carry):
            for k in range(4):
                jj = t * 4 + k
                gwait(k)
                if k == 0:
                    @pl.when(jj > 0)
                    def _():
                        swait(3)
                else:
                    swait(k - 1)

                if k == 3:
                    @pl.when(jj + 3 < NECH)
                    def _():
                        istart(jj + 3, 2)
                else:
                    istart(jj + 3, (k + 3) % 4)
                sstart(k)
                iwait((k + 1) % 4)
                gstart((k + 1) % 4)
            return carry

        lax.fori_loop(0, MAINT, edge_round, 0)

        gwait(0)
        swait(3)
        sstart(0)
        iwait(1)
        gstart(1)
        gwait(1)
        swait(0)
        sstart(1)
        swait(1)
        plsc.subcore_barrier()

        def sloop(k, carry):
            cid = s + k * NS

            @pl.when(cid < NNCH)
            def _():
                pltpu.sync_copy(agg_sh.at[pl.ds(cid * NCHUNK, NCHUNK)], rows0)
                pltpu.sync_copy(inv_hbm.at[pl.ds(cid * NCHUNK, NCHUNK)],
                                inv_v)

                def srow(i, carry2):
                    iv = inv_v[pl.ds(i * LANES, LANES)]
                    for j in range(LANES):
                        m = _lane_bcast(iv, j)
                        r = i * LANES + j
                        rows0[r, :] = rows0[r, :] * m
                    return carry2

                lax.fori_loop(0, NFULL, srow, 0)
                pltpu.sync_copy(rows0, out.at[pl.ds(cid * NCHUNK, NCHUNK)])

            return carry

        lax.fori_loop(0, NROUND, sloop, 0)

    @pl.when(c == 0)
    def _():
        half(tabA, outA)

    @pl.when(c == 1)
    def _():
        half(tabB, outB)


_layer_kernel = pl.kernel(
    _layer_body,
    out_type=(
        jax.ShapeDtypeStruct((NNODES, EMBH), _f32),
        jax.ShapeDtypeStruct((NNODES, EMBH), _f32),
    ),
    mesh=_mesh,
    compiler_params=_params,
    scratch_types=[
        pltpu.VMEM_SHARED((NNODES, EMBH), _f32),
        pltpu.VMEM((2, ECHUNK), _i32),
        pltpu.VMEM((2, ECHUNK), _i32),
        pltpu.VMEM((2, ECHUNK), _i32),
        pltpu.VMEM((2, ECHUNK), _i32),
        pltpu.VMEM((ECHUNK, EMBH), _f32),
        pltpu.VMEM((ECHUNK, EMBH), _f32),
        pltpu.VMEM((NCHUNK,), _f32),
        pltpu.SemaphoreType.DMA,
        pltpu.SemaphoreType.DMA,
        pltpu.SemaphoreType.DMA,
        pltpu.SemaphoreType.DMA,
        pltpu.SemaphoreType.DMA,
        pltpu.SemaphoreType.DMA,
    ],
)


def _final_body(A0, B0, A1, B1, A2, B2, A3, B3, uidx_hbm, iidx_hbm,
                pred_hbm, reg_hbm,
                ids_v, g00, g01, g10, g11, g20, g21, g30, g31,
                pred_v, rbuf_v, sem):
    c = lax.axis_index("c")
    s = lax.axis_index("s")
    w = s * NC + c
    base = w * BPW

    iota = lax.broadcasted_iota(_i32, (LANES,), 0)
    bufs = ((g00, g01), (g10, g11), (g20, g21), (g30, g31))
    tabs = ((A0, B0), (A1, B1), (A2, B2), (A3, B3))

    regacc = jnp.zeros((LANES,), _f32)
    for cc in range(BPW // CHB):
        cbase = base + cc * CHB
        pltpu.sync_copy(uidx_hbm.at[pl.ds(cbase, CHB)],
                        ids_v.at[pl.ds(0, CHB)])
        pltpu.sync_copy(iidx_hbm.at[pl.ds(cbase, CHB)],
                        ids_v.at[pl.ds(CHB, CHB)])

        def addoff(i, carry):
            v = ids_v[pl.ds(CHB + i * LANES, LANES)]
            ids_v[pl.ds(CHB + i * LANES, LANES)] = v + NUSERS
            return carry

        lax.fori_loop(0, CHB // LANES, addoff, 0)

        for l in range(4):
            pltpu.async_copy(tabs[l][0].at[ids_v], bufs[l][0], sem).wait()
            pltpu.async_copy(tabs[l][1].at[ids_v], bufs[l][1], sem).wait()

        def group_body(g, racc):
            pvec = jnp.zeros((LANES,), _f32)
            for j in range(LANES):
                b = g * LANES + j
                uA = g00[b, :] + g10[b, :] + g20[b, :] + g30[b, :]
                uB = g01[b, :] + g11[b, :] + g21[b, :] + g31[b, :]
                iA = (g00[CHB + b, :] + g10[CHB + b, :]
                      + g20[CHB + b, :] + g30[CHB + b, :])
                iB = (g01[CHB + b, :] + g11[CHB + b, :]
                      + g21[CHB + b, :] + g31[CHB + b, :])
                t = uA * iA + uB * iB
                p = jnp.sum(t) * (1.0 / 16.0)
                onehot = (iota == j).astype(_f32)
                pvec = pvec + p * onehot
                u0A = g00[b, :]
                u0B = g01[b, :]
                i0A = g00[CHB + b, :]
                i0B = g01[CHB + b, :]
                racc = (racc + u0A * u0A + u0B * u0B
                        + i0A * i0A + i0B * i0B)
            pred_v[pl.ds(cc * CHB + g * LANES, LANES)] = pvec
            return racc

        regacc = lax.fori_loop(0, CHB // LANES, group_body, regacc)

    pltpu.sync_copy(pred_v, pred_hbm.at[pl.ds(base, BPW)])
    rbuf_v[0, :] = regacc
    pltpu.sync_copy(rbuf_v, reg_hbm.at[pl.ds(w, 1)])


_final_kernel = pl.kernel(
    _final_body,
    out_type=(
        jax.ShapeDtypeStruct((NBATCH,), _f32),
        jax.ShapeDtypeStruct((NW, LANES), _f32),
    ),
    mesh=_mesh,
    compiler_params=_params,
    scratch_types=[
        pltpu.VMEM((2 * CHB,), _i32),
        pltpu.VMEM((2 * CHB, EMBH), _f32),
        pltpu.VMEM((2 * CHB, EMBH), _f32),
        pltpu.VMEM((2 * CHB, EMBH), _f32),
        pltpu.VMEM((2 * CHB, EMBH), _f32),
        pltpu.VMEM((2 * CHB, EMBH), _f32),
        pltpu.VMEM((2 * CHB, EMBH), _f32),
        pltpu.VMEM((2 * CHB, EMBH), _f32),
        pltpu.VMEM((2 * CHB, EMBH), _f32),
        pltpu.VMEM((BPW,), _f32),
        pltpu.VMEM((1, LANES), _f32),
        pltpu.SemaphoreType.DMA,
    ],
)


def kernel(user_indices, item_indices, edge_index, user_table, item_table):
    A0 = jnp.concatenate([user_table[:, :EMBH], item_table[:, :EMBH]], axis=0)
    B0 = jnp.concatenate([user_table[:, EMBH:], item_table[:, EMBH:]], axis=0)
    eidx = edge_index.astype(_i32)
    dst = eidx[1]

    inv = _deg_kernel(dst)
    A1, B1 = _layer_kernel(A0, B0, eidx, inv)
    A2, B2 = _layer_kernel(A1, B1, eidx, inv)
    A3, B3 = _layer_kernel(A2, B2, eidx, inv)
    pred, regpart = _final_kernel(A0, B0, A1, B1, A2, B2, A3, B3,
                                  user_indices.astype(_i32),
                                  item_indices.astype(_i32))
    reg_loss = 0.5 * jnp.sum(regpart) / float(NBATCH)
    return pred, reg_loss


# speedup vs baseline: 22.1676x; 1.0730x over previous
import jax
import jax.numpy as jnp
from jax import lax
from jax.experimental import pallas as pl
from jax.experimental.pallas import tpu as pltpu
from jax.experimental.pallas import tpu_sc as plsc

NUSERS = 50000
NITEMS = 50000
NNODES = 100000
EMBH = 16
NEDGES = 1600000
NBATCH = 16384

NC = 2
NS = 16
NW = NC * NS
LANES = 16

EPT = NEDGES // NS

ECHUNK = 400
NECH = EPT // ECHUNK
MAINT = (NECH - 2) // 4
NCHUNK = 400
NFULL = NCHUNK // LANES
NNCH = NNODES // NCHUNK
NROUND = -(-NNCH // NS)

BPW = NBATCH // NW
CHB = 256

_mesh = plsc.VectorSubcoreMesh(core_axis_name="c", subcore_axis_name="s")
_params = pltpu.CompilerParams(use_tc_tiling_on_sc=False,
                               needs_layout_passes=False)

_f32 = jnp.float32
_i32 = jnp.int32


def _lane_bcast(vec, j):
    return vec.at[jnp.full((LANES,), j, _i32)].get(mode="promise_in_bounds")


def _prop_body(tabA0, tabB0, eidx_hbm,
               outA1, outB1, outA2, outB2, outA3, outB3,
               agg_sh, deg_sh, eb0, eb1, eb2, eb3, rows0, rows1,
               inv_v, ones_v,
               is0, is1, is2, is3, gsem, ssem, dsem):
    c = lax.axis_index("c")
    s = lax.axis_index("s")
    ebufs = (eb0, eb1, eb2, eb3)
    isems = (is0, is1, is2, is3)
    rows = (rows0, rows1)

    def half(hsel):
        tab0 = tabA0 if hsel == 0 else tabB0
        def edge_loop(tab, with_deg):
            def istart(j, slot):
                off = s * EPT + j * ECHUNK
                pltpu.async_copy(eidx_hbm.at[:, pl.ds(off, ECHUNK)],
                                 ebufs[slot], isems[slot])

            def iwait(slot):
                pltpu.make_async_copy(eidx_hbm.at[:, pl.ds(0, ECHUNK)],
                                      ebufs[slot], isems[slot]).wait()

            def gstart(slot):
                pltpu.async_copy(tab.at[ebufs[slot].at[0]], rows[slot % 2],
                                 gsem)

            def gwait(slot):
                pltpu.make_async_copy(tab.at[ebufs[slot].at[0]],
                                      rows[slot % 2], gsem).wait()

            def sstart(slot):
                pltpu.async_copy(rows[slot % 2],
                                 agg_sh.at[ebufs[slot].at[1]],
                                 ssem, add=True)
                if with_deg:
                    pltpu.async_copy(ones_v, deg_sh.at[ebufs[slot].at[1]],
                                     dsem, add=True)

            def swait(slot):
                pltpu.make_async_copy(rows[slot % 2],
                                      agg_sh.at[ebufs[slot].at[1]],
                                      ssem).wait()
                if with_deg:
                    pltpu.make_async_copy(ones_v,
                                          deg_sh.at[ebufs[slot].at[1]],
                                          dsem).wait()

            istart(0, 0)
            istart(1, 1)
            istart(2, 2)
            iwait(0)
            gstart(0)

            def edge_round(t, carry):
                for k in range(4):
                    jj = t * 4 + k
                    gwait(k)
                    if k == 0:
                        @pl.when(jj > 0)
                        def _():
                            swait(3)
                    else:
                        swait(k - 1)

                    if k == 3:
                        @pl.when(jj + 3 < NECH)
                        def _():
                            istart(jj + 3, 2)
                    else:
                        istart(jj + 3, (k + 3) % 4)
                    sstart(k)
                    iwait((k + 1) % 4)
                    gstart((k + 1) % 4)
                return carry

            lax.fori_loop(0, MAINT, edge_round, 0)

            gwait(0)
            swait(3)
            sstart(0)
            iwait(1)
            gstart(1)
            gwait(1)
            swait(0)
            sstart(1)
            swait(1)

        def scale_phase(out, rezero):
            if rezero:
                def fillz1(r, carry):
                    rows1[r, :] = jnp.zeros((LANES,), _f32)
                    return carry

                lax.fori_loop(0, NCHUNK, fillz1, 0)

            def sloop(k, carry):
                cid = s + k * NS

                @pl.when(cid < NNCH)
                def _():
                    pltpu.sync_copy(agg_sh.at[pl.ds(cid * NCHUNK, NCHUNK)],
                                    rows0)
                    pltpu.sync_copy(deg_sh.at[pl.ds(cid * NCHUNK, NCHUNK)],
                                    inv_v)

                    def dloop(i, carry2):
                        v = inv_v[pl.ds(i * LANES, LANES)]
                        inv_v[pl.ds(i * LANES, LANES)] = (
                            1.0 / jnp.maximum(v, 1.0))
                        return carry2

                    lax.fori_loop(0, NFULL, dloop, 0)

                    def srow(i, carry2):
                        iv = inv_v[pl.ds(i * LANES, LANES)]
                        for j in range(LANES):
                            m = _lane_bcast(iv, j)
                            r = i * LANES + j
                            rows0[r, :] = rows0[r, :] * m
                        return carry2

                    lax.fori_loop(0, NFULL, srow, 0)
                    pltpu.sync_copy(rows0,
                                    out.at[pl.ds(cid * NCHUNK, NCHUNK)])
                    if rezero:
                        pltpu.sync_copy(
                            rows1, agg_sh.at[pl.ds(cid * NCHUNK, NCHUNK)])

                return carry

            lax.fori_loop(0, NROUND, sloop, 0)

        def fill1(i, carry):
            ones_v[pl.ds(i * LANES, LANES)] = jnp.full((LANES,), 1.0, _f32)
            return carry

        lax.fori_loop(0, NCHUNK // LANES, fill1, 0)

        def fill0(i, carry):
            inv_v[pl.ds(i * LANES, LANES)] = jnp.zeros((LANES,), _f32)
            return carry

        lax.fori_loop(0, NCHUNK // LANES, fill0, 0)

        def zdeg(k, carry):
            cid = s + k * NS

            @pl.when(cid < NNCH)
            def _():
                pltpu.sync_copy(inv_v, deg_sh.at[pl.ds(cid * NCHUNK, NCHUNK)])

            return carry

        lax.fori_loop(0, NROUND, zdeg, 0)

        def fillz0(r, carry):
            rows0[r, :] = jnp.zeros((LANES,), _f32)
            return carry

        lax.fori_loop(0, NCHUNK, fillz0, 0)

        def zagg(k, carry):
            cid = s + k * NS

            @pl.when(cid < NNCH)
            def _():
                pltpu.sync_copy(rows0, agg_sh.at[pl.ds(cid * NCHUNK, NCHUNK)])

            return carry

        lax.fori_loop(0, NROUND, zagg, 0)
        plsc.subcore_barrier()

        outs = ((outA1, outB1), (outA2, outB2), (outA3, outB3))

        edge_loop(tab0, True)
        plsc.subcore_barrier()
        scale_phase(outs[0][hsel], True)
        plsc.subcore_barrier()

        edge_loop(outs[0][hsel], False)
        plsc.subcore_barrier()
        scale_phase(outs[1][hsel], True)
        plsc.subcore_barrier()

        edge_loop(outs[1][hsel], False)
        plsc.subcore_barrier()
        scale_phase(outs[2][hsel], False)

    @pl.when(c == 0)
    def _():
        half(0)

    @pl.when(c == 1)
    def _():
        half(1)


_prop_kernel = pl.kernel(
    _prop_body,
    out_type=tuple(jax.ShapeDtypeStruct((NNODES, EMBH), _f32)
                   for _ in range(6)),
    mesh=_mesh,
    compiler_params=_params,
    scratch_types=[
        pltpu.VMEM_SHARED((NNODES, EMBH), _f32),
        pltpu.VMEM_SHARED((NNODES,), _f32),
        pltpu.VMEM((2, ECHUNK), _i32),
        pltpu.VMEM((2, ECHUNK), _i32),
        pltpu.VMEM((2, ECHUNK), _i32),
        pltpu.VMEM((2, ECHUNK), _i32),
        pltpu.VMEM((ECHUNK, EMBH), _f32),
        pltpu.VMEM((ECHUNK, EMBH), _f32),
        pltpu.VMEM((NCHUNK,), _f32),
        pltpu.VMEM((ECHUNK,), _f32),
        pltpu.SemaphoreType.DMA,
        pltpu.SemaphoreType.DMA,
        pltpu.SemaphoreType.DMA,
        pltpu.SemaphoreType.DMA,
        pltpu.SemaphoreType.DMA,
        pltpu.SemaphoreType.DMA,
        pltpu.SemaphoreType.DMA,
    ],
)


def _final_body(A0, B0, A1, B1, A2, B2, A3, B3, uidx_hbm, iidx_hbm,
                pred_hbm, reg_hbm,
                ids_v, g00, g01, g10, g11, g20, g21, g30, g31,
                pred_v, rbuf_v, sem):
    c = lax.axis_index("c")
    s = lax.axis_index("s")
    w = s * NC + c
    base = w * BPW

    iota = lax.broadcasted_iota(_i32, (LANES,), 0)
    bufs = ((g00, g01), (g10, g11), (g20, g21), (g30, g31))
    tabs = ((A0, B0), (A1, B1), (A2, B2), (A3, B3))

    regacc = jnp.zeros((LANES,), _f32)
    for cc in range(BPW // CHB):
        cbase = base + cc * CHB
        pltpu.sync_copy(uidx_hbm.at[pl.ds(cbase, CHB)],
                        ids_v.at[pl.ds(0, CHB)])
        pltpu.sync_copy(iidx_hbm.at[pl.ds(cbase, CHB)],
                        ids_v.at[pl.ds(CHB, CHB)])

        def addoff(i, carry):
            v = ids_v[pl.ds(CHB + i * LANES, LANES)]
            ids_v[pl.ds(CHB + i * LANES, LANES)] = v + NUSERS
            return carry

        lax.fori_loop(0, CHB // LANES, addoff, 0)

        for l in range(4):
            pltpu.async_copy(tabs[l][0].at[ids_v], bufs[l][0], sem).wait()
            pltpu.async_copy(tabs[l][1].at[ids_v], bufs[l][1], sem).wait()

        def group_body(g, racc):
            pvec = jnp.zeros((LANES,), _f32)
            for j in range(LANES):
                b = g * LANES + j
                uA = g00[b, :] + g10[b, :] + g20[b, :] + g30[b, :]
                uB = g01[b, :] + g11[b, :] + g21[b, :] + g31[b, :]
                iA = (g00[CHB + b, :] + g10[CHB + b, :]
                      + g20[CHB + b, :] + g30[CHB + b, :])
                iB = (g01[CHB + b, :] + g11[CHB + b, :]
                      + g21[CHB + b, :] + g31[CHB + b, :])
                t = uA * iA + uB * iB
                p = jnp.sum(t) * (1.0 / 16.0)
                onehot = (iota == j).astype(_f32)
                pvec = pvec + p * onehot
                u0A = g00[b, :]
                u0B = g01[b, :]
                i0A = g00[CHB + b, :]
                i0B = g01[CHB + b, :]
                racc = (racc + u0A * u0A + u0B * u0B
                        + i0A * i0A + i0B * i0B)
            pred_v[pl.ds(cc * CHB + g * LANES, LANES)] = pvec
            return racc

        regacc = lax.fori_loop(0, CHB // LANES, group_body, regacc)

    pltpu.sync_copy(pred_v, pred_hbm.at[pl.ds(base, BPW)])
    rbuf_v[0, :] = regacc
    pltpu.sync_copy(rbuf_v, reg_hbm.at[pl.ds(w, 1)])


_final_kernel = pl.kernel(
    _final_body,
    out_type=(
        jax.ShapeDtypeStruct((NBATCH,), _f32),
        jax.ShapeDtypeStruct((NW, LANES), _f32),
    ),
    mesh=_mesh,
    compiler_params=_params,
    scratch_types=[
        pltpu.VMEM((2 * CHB,), _i32),
        pltpu.VMEM((2 * CHB, EMBH), _f32),
        pltpu.VMEM((2 * CHB, EMBH), _f32),
        pltpu.VMEM((2 * CHB, EMBH), _f32),
        pltpu.VMEM((2 * CHB, EMBH), _f32),
        pltpu.VMEM((2 * CHB, EMBH), _f32),
        pltpu.VMEM((2 * CHB, EMBH), _f32),
        pltpu.VMEM((2 * CHB, EMBH), _f32),
        pltpu.VMEM((2 * CHB, EMBH), _f32),
        pltpu.VMEM((BPW,), _f32),
        pltpu.VMEM((1, LANES), _f32),
        pltpu.SemaphoreType.DMA,
    ],
)


def kernel(user_indices, item_indices, edge_index, user_table, item_table):
    A0 = jnp.concatenate([user_table[:, :EMBH], item_table[:, :EMBH]], axis=0)
    B0 = jnp.concatenate([user_table[:, EMBH:], item_table[:, EMBH:]], axis=0)
    eidx = edge_index.astype(_i32)

    A1, B1, A2, B2, A3, B3 = _prop_kernel(A0, B0, eidx)
    pred, regpart = _final_kernel(A0, B0, A1, B1, A2, B2, A3, B3,
                                  user_indices.astype(_i32),
                                  item_indices.astype(_i32))
    reg_loss = 0.5 * jnp.sum(regpart) / float(NBATCH)
    return pred, reg_loss


# speedup vs baseline: 24.1220x; 1.0882x over previous
import jax
import jax.numpy as jnp
from jax import lax
from jax.experimental import pallas as pl
from jax.experimental.pallas import tpu as pltpu
from jax.experimental.pallas import tpu_sc as plsc

NUSERS = 50000
NITEMS = 50000
NNODES = 100000
EMBH = 16
NEDGES = 1600000
NBATCH = 16384

NC = 2
NS = 16
NW = NC * NS
LANES = 16

EPT = NEDGES // NS

ECHUNK = 200
NECH = EPT // ECHUNK
MAINT = (NECH - 4) // 4
NCHUNK = 200
NFULL = NCHUNK // LANES
NTAIL = NCHUNK - NFULL * LANES
IPAD = 208
NNCH = NNODES // NCHUNK
NROUND = -(-NNCH // NS)

BPW = NBATCH // NW
CHB = 256

_mesh = plsc.VectorSubcoreMesh(core_axis_name="c", subcore_axis_name="s")
_params = pltpu.CompilerParams(use_tc_tiling_on_sc=False,
                               needs_layout_passes=False)

_f32 = jnp.float32
_i32 = jnp.int32


def _lane_bcast(vec, j):
    return vec.at[jnp.full((LANES,), j, _i32)].get(mode="promise_in_bounds")


def _prop_body(tabA0, tabB0, eidx_hbm,
               outA1, outB1, outA2, outB2, outA3, outB3,
               agg_sh, deg_sh, eb0, eb1, eb2, eb3,
               rows0, rows1, rows2, rows3,
               inv_v, ones_v,
               is0, is1, is2, is3, gs0, gs1, gs2, gs3,
               ss0, ss1, ss2, ss3, dsem):
    c = lax.axis_index("c")
    s = lax.axis_index("s")
    ebufs = (eb0, eb1, eb2, eb3)
    isems = (is0, is1, is2, is3)
    gsems = (gs0, gs1, gs2, gs3)
    ssems = (ss0, ss1, ss2, ss3)
    rows = (rows0, rows1, rows2, rows3)

    def half(hsel):
        tab0 = tabA0 if hsel == 0 else tabB0
        def edge_loop(tab, with_deg):
            def istart(j, slot):
                off = s * EPT + j * ECHUNK
                pltpu.async_copy(eidx_hbm.at[:, pl.ds(off, ECHUNK)],
                                 ebufs[slot], isems[slot])

            def iwait(slot):
                pltpu.make_async_copy(eidx_hbm.at[:, pl.ds(0, ECHUNK)],
                                      ebufs[slot], isems[slot]).wait()

            def gstart(slot):
                pltpu.async_copy(tab.at[ebufs[slot].at[0]], rows[slot],
                                 gsems[slot])

            def gwait(slot):
                pltpu.make_async_copy(tab.at[ebufs[slot].at[0]],
                                      rows[slot], gsems[slot]).wait()

            def sstart(slot):
                pltpu.async_copy(rows[slot],
                                 agg_sh.at[ebufs[slot].at[1]],
                                 ssems[slot], add=True)
                if with_deg:
                    pltpu.async_copy(ones_v.at[pl.ds(0, ECHUNK)],
                                     deg_sh.at[ebufs[slot].at[1]],
                                     dsem, add=True)

            def swait(slot):
                pltpu.make_async_copy(rows[slot],
                                      agg_sh.at[ebufs[slot].at[1]],
                                      ssems[slot]).wait()
                if with_deg:
                    pltpu.make_async_copy(ones_v.at[pl.ds(0, ECHUNK)],
                                          deg_sh.at[ebufs[slot].at[1]],
                                          dsem).wait()

            istart(0, 0)
            istart(1, 1)
            istart(2, 2)
            iwait(0)
            gstart(0)
            iwait(1)
            gstart(1)

            def edge_round(t, carry):
                for k in range(4):
                    jj = t * 4 + k
                    gwait(k)
                    if k == 0:
                        @pl.when(jj > 0)
                        def _():
                            swait(3)
                    else:
                        swait(k - 1)
                    istart(jj + 3, (k + 3) % 4)
                    sstart(k)
                    iwait((k + 2) % 4)
                    gstart((k + 2) % 4)
                return carry

            lax.fori_loop(0, MAINT, edge_round, 0)

            gwait(0)
            swait(3)
            istart(NECH - 1, 3)
            sstart(0)
            iwait(2)
            gstart(2)

            gwait(1)
            swait(0)
            sstart(1)
            iwait(3)
            gstart(3)

            gwait(2)
            swait(1)
            sstart(2)

            gwait(3)
            swait(2)
            sstart(3)
            swait(3)

        def scale_phase(out, rezero):
            if rezero:
                def fillz1(r, carry):
                    rows1[r, :] = jnp.zeros((LANES,), _f32)
                    return carry

                lax.fori_loop(0, NCHUNK, fillz1, 0)

            def sloop(k, carry):
                cid = s + k * NS

                @pl.when(cid < NNCH)
                def _():
                    pltpu.sync_copy(agg_sh.at[pl.ds(cid * NCHUNK, NCHUNK)],
                                    rows0)
                    pltpu.sync_copy(deg_sh.at[pl.ds(cid * NCHUNK, NCHUNK)],
                                    inv_v.at[pl.ds(0, NCHUNK)])

                    def dloop(i, carry2):
                        v = inv_v[pl.ds(i * LANES, LANES)]
                        inv_v[pl.ds(i * LANES, LANES)] = (
                            1.0 / jnp.maximum(v, 1.0))
                        return carry2

                    lax.fori_loop(0, IPAD // LANES, dloop, 0)

                    def srow(i, carry2):
                        iv = inv_v[pl.ds(i * LANES, LANES)]
                        for j in range(LANES):
                            m = _lane_bcast(iv, j)
                            r = i * LANES + j
                            rows0[r, :] = rows0[r, :] * m
                        return carry2

                    lax.fori_loop(0, NFULL, srow, 0)
                    iv_t = inv_v[pl.ds(NFULL * LANES, LANES)]
                    for j in range(NTAIL):
                        m = _lane_bcast(iv_t, j)
                        r = NFULL * LANES + j
                        rows0[r, :] = rows0[r, :] * m
                    pltpu.sync_copy(rows0,
                                    out.at[pl.ds(cid * NCHUNK, NCHUNK)])
                    if rezero:
                        pltpu.sync_copy(
                            rows1, agg_sh.at[pl.ds(cid * NCHUNK, NCHUNK)])

                return carry

            lax.fori_loop(0, NROUND, sloop, 0)

        def fill1(i, carry):
            ones_v[pl.ds(i * LANES, LANES)] = jnp.full((LANES,), 1.0, _f32)
            return carry

        lax.fori_loop(0, IPAD // LANES, fill1, 0)

        def fill0(i, carry):
            inv_v[pl.ds(i * LANES, LANES)] = jnp.zeros((LANES,), _f32)
            return carry

        lax.fori_loop(0, IPAD // LANES, fill0, 0)

        def zdeg(k, carry):
            cid = s + k * NS

            @pl.when(cid < NNCH)
            def _():
                pltpu.sync_copy(inv_v.at[pl.ds(0, NCHUNK)],
                                deg_sh.at[pl.ds(cid * NCHUNK, NCHUNK)])

            return carry

        lax.fori_loop(0, NROUND, zdeg, 0)

        def fillz0(r, carry):
            rows0[r, :] = jnp.zeros((LANES,), _f32)
            return carry

        lax.fori_loop(0, NCHUNK, fillz0, 0)

        def zagg(k, carry):
            cid = s + k * NS

            @pl.when(cid < NNCH)
            def _():
                pltpu.sync_copy(rows0, agg_sh.at[pl.ds(cid * NCHUNK, NCHUNK)])

            return carry

        lax.fori_loop(0, NROUND, zagg, 0)
        plsc.subcore_barrier()

        outs = ((outA1, outB1), (outA2, outB2), (outA3, outB3))

        edge_loop(tab0, True)
        plsc.subcore_barrier()
        scale_phase(outs[0][hsel], True)
        plsc.subcore_barrier()

        edge_loop(outs[0][hsel], False)
        plsc.subcore_barrier()
        scale_phase(outs[1][hsel], True)
        plsc.subcore_barrier()

        edge_loop(outs[1][hsel], False)
        plsc.subcore_barrier()
        scale_phase(outs[2][hsel], False)

    @pl.when(c == 0)
    def _():
        half(0)

    @pl.when(c == 1)
    def _():
        half(1)


_prop_kernel = pl.kernel(
    _prop_body,
    out_type=tuple(jax.ShapeDtypeStruct((NNODES, EMBH), _f32)
                   for _ in range(6)),
    mesh=_mesh,
    compiler_params=_params,
    scratch_types=[
        pltpu.VMEM_SHARED((NNODES, EMBH), _f32),
        pltpu.VMEM_SHARED((NNODES,), _f32),
        pltpu.VMEM((2, ECHUNK), _i32),
        pltpu.VMEM((2, ECHUNK), _i32),
        pltpu.VMEM((2, ECHUNK), _i32),
        pltpu.VMEM((2, ECHUNK), _i32),
        pltpu.VMEM((ECHUNK, EMBH), _f32),
        pltpu.VMEM((ECHUNK, EMBH), _f32),
        pltpu.VMEM((ECHUNK, EMBH), _f32),
        pltpu.VMEM((ECHUNK, EMBH), _f32),
        pltpu.VMEM((IPAD,), _f32),
        pltpu.VMEM((IPAD,), _f32),
    ] + [pltpu.SemaphoreType.DMA] * 13,
)


def _final_body(A0, B0, A1, B1, A2, B2, A3, B3, uidx_hbm, iidx_hbm,
                pred_hbm, reg_hbm,
                ids_v, g00, g01, g10, g11, g20, g21, g30, g31,
                pred_v, rbuf_v, sem):
    c = lax.axis_index("c")
    s = lax.axis_index("s")
    w = s * NC + c
    base = w * BPW

    iota = lax.broadcasted_iota(_i32, (LANES,), 0)
    bufs = ((g00, g01), (g10, g11), (g20, g21), (g30, g31))
    tabs = ((A0, B0), (A1, B1), (A2, B2), (A3, B3))

    regacc = jnp.zeros((LANES,), _f32)
    for cc in range(BPW // CHB):
        cbase = base + cc * CHB
        pltpu.sync_copy(uidx_hbm.at[pl.ds(cbase, CHB)],
                        ids_v.at[pl.ds(0, CHB)])
        pltpu.sync_copy(iidx_hbm.at[pl.ds(cbase, CHB)],
                        ids_v.at[pl.ds(CHB, CHB)])

        def addoff(i, carry):
            v = ids_v[pl.ds(CHB + i * LANES, LANES)]
            ids_v[pl.ds(CHB + i * LANES, LANES)] = v + NUSERS
            return carry

        lax.fori_loop(0, CHB // LANES, addoff, 0)

        for l in range(4):
            pltpu.async_copy(tabs[l][0].at[ids_v], bufs[l][0], sem).wait()
            pltpu.async_copy(tabs[l][1].at[ids_v], bufs[l][1], sem).wait()

        def group_body(g, racc):
            pvec = jnp.zeros((LANES,), _f32)
            for j in range(LANES):
                b = g * LANES + j
                uA = g00[b, :] + g10[b, :] + g20[b, :] + g30[b, :]
                uB = g01[b, :] + g11[b, :] + g21[b, :] + g31[b, :]
                iA = (g00[CHB + b, :] + g10[CHB + b, :]
                      + g20[CHB + b, :] + g30[CHB + b, :])
                iB = (g01[CHB + b, :] + g11[CHB + b, :]
                      + g21[CHB + b, :] + g31[CHB + b, :])
                t = uA * iA + uB * iB
                p = jnp.sum(t) * (1.0 / 16.0)
                onehot = (iota == j).astype(_f32)
                pvec = pvec + p * onehot
                u0A = g00[b, :]
                u0B = g01[b, :]
                i0A = g00[CHB + b, :]
                i0B = g01[CHB + b, :]
                racc = (racc + u0A * u0A + u0B * u0B
                        + i0A * i0A + i0B * i0B)
            pred_v[pl.ds(cc * CHB + g * LANES, LANES)] = pvec
            return racc

        regacc = lax.fori_loop(0, CHB // LANES, group_body, regacc)

    pltpu.sync_copy(pred_v, pred_hbm.at[pl.ds(base, BPW)])
    rbuf_v[0, :] = regacc
    pltpu.sync_copy(rbuf_v, reg_hbm.at[pl.ds(w, 1)])


_final_kernel = pl.kernel(
    _final_body,
    out_type=(
        jax.ShapeDtypeStruct((NBATCH,), _f32),
        jax.ShapeDtypeStruct((NW, LANES), _f32),
    ),
    mesh=_mesh,
    compiler_params=_params,
    scratch_types=[
        pltpu.VMEM((2 * CHB,), _i32),
        pltpu.VMEM((2 * CHB, EMBH), _f32),
        pltpu.VMEM((2 * CHB, EMBH), _f32),
        pltpu.VMEM((2 * CHB, EMBH), _f32),
        pltpu.VMEM((2 * CHB, EMBH), _f32),
        pltpu.VMEM((2 * CHB, EMBH), _f32),
        pltpu.VMEM((2 * CHB, EMBH), _f32),
        pltpu.VMEM((2 * CHB, EMBH), _f32),
        pltpu.VMEM((2 * CHB, EMBH), _f32),
        pltpu.VMEM((BPW,), _f32),
        pltpu.VMEM((1, LANES), _f32),
        pltpu.SemaphoreType.DMA,
    ],
)


def kernel(user_indices, item_indices, edge_index, user_table, item_table):
    A0 = jnp.concatenate([user_table[:, :EMBH], item_table[:, :EMBH]], axis=0)
    B0 = jnp.concatenate([user_table[:, EMBH:], item_table[:, EMBH:]], axis=0)
    eidx = edge_index.astype(_i32)

    A1, B1, A2, B2, A3, B3 = _prop_kernel(A0, B0, eidx)
    pred, regpart = _final_kernel(A0, B0, A1, B1, A2, B2, A3, B3,
                                  user_indices.astype(_i32),
                                  item_indices.astype(_i32))
    reg_loss = 0.5 * jnp.sum(regpart) / float(NBATCH)
    return pred, reg_loss


# speedup vs baseline: 29.0984x; 1.2063x over previous
import jax
import jax.numpy as jnp
from jax import lax
from jax.experimental import pallas as pl
from jax.experimental.pallas import tpu as pltpu
from jax.experimental.pallas import tpu_sc as plsc

NUSERS = 50000
NITEMS = 50000
NNODES = 100000
EMBH = 16
NEDGES = 1600000
NBATCH = 16384

NC = 2
NS = 16
NW = NC * NS
LANES = 16

EPT = NEDGES // NS

ECHUNK = 200
NECH = EPT // ECHUNK
MAINT = (NECH - 5) // 5
NCHUNK = 200
NFULL = NCHUNK // LANES
NTAIL = NCHUNK - NFULL * LANES
IPAD = 208
NNCH = NNODES // NCHUNK
NROUND = -(-NNCH // NS)

BPW = NBATCH // NW
CHB = 256

_mesh = plsc.VectorSubcoreMesh(core_axis_name="c", subcore_axis_name="s")
_params = pltpu.CompilerParams(use_tc_tiling_on_sc=False,
                               needs_layout_passes=False)

_f32 = jnp.float32
_i32 = jnp.int32


def _lane_bcast(vec, j):
    return vec.at[jnp.full((LANES,), j, _i32)].get(mode="promise_in_bounds")


def _prop_body(tabA0, tabB0, eidx_hbm,
               outA1, outB1, outA2, outB2, outA3, outB3,
               agg_sh, deg_sh, eb0, eb1, eb2, eb3, eb4,
               rows0, rows1, rows2, rows3, rows4,
               inv_v, ones_v,
               is0, is1, is2, is3, is4, gs0, gs1, gs2, gs3, gs4,
               ss0, ss1, ss2, ss3, ss4, dsem):
    c = lax.axis_index("c")
    s = lax.axis_index("s")
    ebufs = (eb0, eb1, eb2, eb3, eb4)
    isems = (is0, is1, is2, is3, is4)
    gsems = (gs0, gs1, gs2, gs3, gs4)
    ssems = (ss0, ss1, ss2, ss3, ss4)
    rows = (rows0, rows1, rows2, rows3, rows4)

    def half(hsel):
        tab0 = tabA0 if hsel == 0 else tabB0
        def edge_loop(tab, with_deg):
            def istart(j, slot):
                off = s * EPT + j * ECHUNK
                pltpu.async_copy(eidx_hbm.at[:, pl.ds(off, ECHUNK)],
                                 ebufs[slot], isems[slot])

            def iwait(slot):
                pltpu.make_async_copy(eidx_hbm.at[:, pl.ds(0, ECHUNK)],
                                      ebufs[slot], isems[slot]).wait()

            def gstart(slot):
                pltpu.async_copy(tab.at[ebufs[slot].at[0]], rows[slot],
                                 gsems[slot])

            def gwait(slot):
                pltpu.make_async_copy(tab.at[ebufs[slot].at[0]],
                                      rows[slot], gsems[slot]).wait()

            def sstart(slot):
                pltpu.async_copy(rows[slot],
                                 agg_sh.at[ebufs[slot].at[1]],
                                 ssems[slot], add=True)
                if with_deg:
                    pltpu.async_copy(ones_v.at[pl.ds(0, ECHUNK)],
                                     deg_sh.at[ebufs[slot].at[1]],
                                     dsem, add=True)

            def swait(slot):
                pltpu.make_async_copy(rows[slot],
                                      agg_sh.at[ebufs[slot].at[1]],
                                      ssems[slot]).wait()
                if with_deg:
                    pltpu.make_async_copy(ones_v.at[pl.ds(0, ECHUNK)],
                                          deg_sh.at[ebufs[slot].at[1]],
                                          dsem).wait()

            istart(0, 0)
            istart(1, 1)
            istart(2, 2)
            istart(3, 3)
            iwait(0)
            gstart(0)
            iwait(1)
            gstart(1)
            iwait(2)
            gstart(2)

            def edge_round(t, carry):
                for k in range(5):
                    jj = t * 5 + k
                    gwait(k)
                    if k == 0:
                        @pl.when(jj > 0)
                        def _():
                            swait(4)
                    else:
                        swait(k - 1)
                    istart(jj + 4, (k + 4) % 5)
                    sstart(k)
                    iwait((k + 3) % 5)
                    gstart((k + 3) % 5)
                return carry

            lax.fori_loop(0, MAINT, edge_round, 0)

            gwait(0)
            swait(4)
            istart(NECH - 1, 4)
            sstart(0)
            iwait(3)
            gstart(3)

            gwait(1)
            swait(0)
            sstart(1)
            iwait(4)
            gstart(4)

            gwait(2)
            swait(1)
            sstart(2)

            gwait(3)
            swait(2)
            sstart(3)

            gwait(4)
            swait(3)
            sstart(4)
            swait(4)

        def scale_phase(out, rezero):
            if rezero:
                def fillz1(r, carry):
                    rows1[r, :] = jnp.zeros((LANES,), _f32)
                    return carry

                lax.fori_loop(0, NCHUNK, fillz1, 0)

            def sloop(k, carry):
                cid = s + k * NS

                @pl.when(cid < NNCH)
                def _():
                    pltpu.sync_copy(agg_sh.at[pl.ds(cid * NCHUNK, NCHUNK)],
                                    rows0)
                    pltpu.sync_copy(deg_sh.at[pl.ds(cid * NCHUNK, NCHUNK)],
                                    inv_v.at[pl.ds(0, NCHUNK)])

                    def dloop(i, carry2):
                        v = inv_v[pl.ds(i * LANES, LANES)]
                        inv_v[pl.ds(i * LANES, LANES)] = (
                            1.0 / jnp.maximum(v, 1.0))
                        return carry2

                    lax.fori_loop(0, IPAD // LANES, dloop, 0)

                    def srow(i, carry2):
                        iv = inv_v[pl.ds(i * LANES, LANES)]
                        for j in range(LANES):
                            m = _lane_bcast(iv, j)
                            r = i * LANES + j
                            rows0[r, :] = rows0[r, :] * m
                        return carry2

                    lax.fori_loop(0, NFULL, srow, 0)
                    iv_t = inv_v[pl.ds(NFULL * LANES, LANES)]
                    for j in range(NTAIL):
                        m = _lane_bcast(iv_t, j)
                        r = NFULL * LANES + j
                        rows0[r, :] = rows0[r, :] * m
                    pltpu.sync_copy(rows0,
                                    out.at[pl.ds(cid * NCHUNK, NCHUNK)])
                    if rezero:
                        pltpu.sync_copy(
                            rows1, agg_sh.at[pl.ds(cid * NCHUNK, NCHUNK)])

                return carry

            lax.fori_loop(0, NROUND, sloop, 0)

        def fill1(i, carry):
            ones_v[pl.ds(i * LANES, LANES)] = jnp.full((LANES,), 1.0, _f32)
            return carry

        lax.fori_loop(0, IPAD // LANES, fill1, 0)

        def fill0(i, carry):
            inv_v[pl.ds(i * LANES, LANES)] = jnp.zeros((LANES,), _f32)
            return carry

        lax.fori_loop(0, IPAD // LANES, fill0, 0)

        def zdeg(k, carry):
            cid = s + k * NS

            @pl.when(cid < NNCH)
            def _():
                pltpu.sync_copy(inv_v.at[pl.ds(0, NCHUNK)],
                                deg_sh.at[pl.ds(cid * NCHUNK, NCHUNK)])

            return carry

        lax.fori_loop(0, NROUND, zdeg, 0)

        def fillz0(r, carry):
            rows0[r, :] = jnp.zeros((LANES,), _f32)
            return carry

        lax.fori_loop(0, NCHUNK, fillz0, 0)

        def zagg(k, carry):
            cid = s + k * NS

            @pl.when(cid < NNCH)
            def _():
                pltpu.sync_copy(rows0, agg_sh.at[pl.ds(cid * NCHUNK, NCHUNK)])

            return carry

        lax.fori_loop(0, NROUND, zagg, 0)
        plsc.subcore_barrier()

        outs = ((outA1, outB1), (outA2, outB2), (outA3, outB3))

        with jax.named_scope("edge1"):
            edge_loop(tab0, True)
            plsc.subcore_barrier()
        with jax.named_scope("scale1"):
            scale_phase(outs[0][hsel], True)
            plsc.subcore_barrier()

        with jax.named_scope("edge2"):
            edge_loop(outs[0][hsel], False)
            plsc.subcore_barrier()
        with jax.named_scope("scale2"):
            scale_phase(outs[1][hsel], True)
            plsc.subcore_barrier()

        with jax.named_scope("edge3"):
            edge_loop(outs[1][hsel], False)
            plsc.subcore_barrier()
        with jax.named_scope("scale3"):
            scale_phase(outs[2][hsel], False)

    @pl.when(c == 0)
    def _():
        half(0)

    @pl.when(c == 1)
    def _():
        half(1)


_prop_kernel = pl.kernel(
    _prop_body,
    out_type=tuple(jax.ShapeDtypeStruct((NNODES, EMBH), _f32)
                   for _ in range(6)),
    mesh=_mesh,
    compiler_params=_params,
    scratch_types=[
        pltpu.VMEM_SHARED((NNODES, EMBH), _f32),
        pltpu.VMEM_SHARED((NNODES,), _f32),
        pltpu.VMEM((2, ECHUNK), _i32),
        pltpu.VMEM((2, ECHUNK), _i32),
        pltpu.VMEM((2, ECHUNK), _i32),
        pltpu.VMEM((2, ECHUNK), _i32),
        pltpu.VMEM((2, ECHUNK), _i32),
        pltpu.VMEM((ECHUNK, EMBH), _f32),
        pltpu.VMEM((ECHUNK, EMBH), _f32),
        pltpu.VMEM((ECHUNK, EMBH), _f32),
        pltpu.VMEM((ECHUNK, EMBH), _f32),
        pltpu.VMEM((ECHUNK, EMBH), _f32),
        pltpu.VMEM((IPAD,), _f32),
        pltpu.VMEM((IPAD,), _f32),
    ] + [pltpu.SemaphoreType.DMA] * 16,
)


def _final_body(A0, B0, A1, B1, A2, B2, A3, B3, uidx_hbm, iidx_hbm,
                pred_hbm, reg_hbm,
                ids_v, g00, g01, g10, g11, g20, g21, g30, g31,
                pred_v, rbuf_v, sem):
    c = lax.axis_index("c")
    s = lax.axis_index("s")
    w = s * NC + c
    base = w * BPW

    iota = lax.broadcasted_iota(_i32, (LANES,), 0)
    bufs = ((g00, g01), (g10, g11), (g20, g21), (g30, g31))
    tabs = ((A0, B0), (A1, B1), (A2, B2), (A3, B3))

    regacc = jnp.zeros((LANES,), _f32)
    for cc in range(BPW // CHB):
        cbase = base + cc * CHB
        pltpu.sync_copy(uidx_hbm.at[pl.ds(cbase, CHB)],
                        ids_v.at[pl.ds(0, CHB)])
        pltpu.sync_copy(iidx_hbm.at[pl.ds(cbase, CHB)],
                        ids_v.at[pl.ds(CHB, CHB)])

        def addoff(i, carry):
            v = ids_v[pl.ds(CHB + i * LANES, LANES)]
            ids_v[pl.ds(CHB + i * LANES, LANES)] = v + NUSERS
            return carry

        lax.fori_loop(0, CHB // LANES, addoff, 0)

        for l in range(4):
            pltpu.async_copy(tabs[l][0].at[ids_v], bufs[l][0], sem).wait()
            pltpu.async_copy(tabs[l][1].at[ids_v], bufs[l][1], sem).wait()

        def group_body(g, racc):
            pvec = jnp.zeros((LANES,), _f32)
            for j in range(LANES):
                b = g * LANES + j
                uA = g00[b, :] + g10[b, :] + g20[b, :] + g30[b, :]
                uB = g01[b, :] + g11[b, :] + g21[b, :] + g31[b, :]
                iA = (g00[CHB + b, :] + g10[CHB + b, :]
                      + g20[CHB + b, :] + g30[CHB + b, :])
                iB = (g01[CHB + b, :] + g11[CHB + b, :]
                      + g21[CHB + b, :] + g31[CHB + b, :])
                t = uA * iA + uB * iB
                p = jnp.sum(t) * (1.0 / 16.0)
                onehot = (iota == j).astype(_f32)
                pvec = pvec + p * onehot
                u0A = g00[b, :]
                u0B = g01[b, :]
                i0A = g00[CHB + b, :]
                i0B = g01[CHB + b, :]
                racc = (racc + u0A * u0A + u0B * u0B
                        + i0A * i0A + i0B * i0B)
            pred_v[pl.ds(cc * CHB + g * LANES, LANES)] = pvec
            return racc

        regacc = lax.fori_loop(0, CHB // LANES, group_body, regacc)

    pltpu.sync_copy(pred_v, pred_hbm.at[pl.ds(base, BPW)])
    rbuf_v[0, :] = regacc
    pltpu.sync_copy(rbuf_v, reg_hbm.at[pl.ds(w, 1)])


_final_kernel = pl.kernel(
    _final_body,
    out_type=(
        jax.ShapeDtypeStruct((NBATCH,), _f32),
        jax.ShapeDtypeStruct((NW, LANES), _f32),
    ),
    mesh=_mesh,
    compiler_params=_params,
    scratch_types=[
        pltpu.VMEM((2 * CHB,), _i32),
        pltpu.VMEM((2 * CHB, EMBH), _f32),
        pltpu.VMEM((2 * CHB, EMBH), _f32),
        pltpu.VMEM((2 * CHB, EMBH), _f32),
        pltpu.VMEM((2 * CHB, EMBH), _f32),
        pltpu.VMEM((2 * CHB, EMBH), _f32),
        pltpu.VMEM((2 * CHB, EMBH), _f32),
        pltpu.VMEM((2 * CHB, EMBH), _f32),
        pltpu.VMEM((2 * CHB, EMBH), _f32),
        pltpu.VMEM((BPW,), _f32),
        pltpu.VMEM((1, LANES), _f32),
        pltpu.SemaphoreType.DMA,
    ],
)


def kernel(user_indices, item_indices, edge_index, user_table, item_table):
    A0 = jnp.concatenate([user_table[:, :EMBH], item_table[:, :EMBH]], axis=0)
    B0 = jnp.concatenate([user_table[:, EMBH:], item_table[:, EMBH:]], axis=0)
    eidx = edge_index.astype(_i32)

    A1, B1, A2, B2, A3, B3 = _prop_kernel(A0, B0, eidx)
    pred, regpart = _final_kernel(A0, B0, A1, B1, A2, B2, A3, B3,
                                  user_indices.astype(_i32),
                                  item_indices.astype(_i32))
    reg_loss = 0.5 * jnp.sum(regpart) / float(NBATCH)
    return pred, reg_loss
